# baseline (speedup 1.0000x reference)
import functools

import jax
import jax.numpy as jnp
from jax import lax
from jax.experimental import pallas as pl
from jax.experimental.pallas import tpu as pltpu
from jax.experimental.pallas import tpu_sc as plsc

VOCAB = 100000
MODEL_DIM = 64
BATCH = 16384

NUM_CORES = 2
NUM_SUBCORES = 16
NUM_WORKERS = NUM_CORES * NUM_SUBCORES
B_PER_W = BATCH // NUM_WORKERS
IDX_CHUNK = 128
NUM_CHUNKS = B_PER_W // IDX_CHUNK
LANES = 16
ROUND_MAGIC = 12582912.0
QMIN = -127.0
QMAX = 127.0


def _quantize(v, sv):
    q = (v + ROUND_MAGIC) - ROUND_MAGIC
    q = jnp.minimum(jnp.maximum(q, QMIN), QMAX)
    return q * sv


@functools.partial(jax.jit, static_argnames=())
def _embed(x, weights, scales):
    mesh = plsc.VectorSubcoreMesh(core_axis_name="c", subcore_axis_name="s")

    @functools.partial(
        pl.kernel,
        mesh=mesh,
        out_type=jax.ShapeDtypeStruct((BATCH, MODEL_DIM), jnp.float32),
        scratch_types=[
            pltpu.VMEM((B_PER_W,), jnp.int32),
            pltpu.VMEM((B_PER_W, MODEL_DIM), jnp.float32),
            pltpu.VMEM((B_PER_W,), jnp.float32),
            pltpu.SemaphoreType.DMA,
        ],
        compiler_params=pltpu.CompilerParams(use_tc_tiling_on_sc=False),
    )
    def k(x_hbm, w_hbm, s_hbm, out_hbm, idx_v, rows_v, scl_v, sem):
        wid = lax.axis_index("s") * NUM_CORES + lax.axis_index("c")
        base = wid * B_PER_W
        pltpu.sync_copy(x_hbm.at[pl.ds(base, B_PER_W)], idx_v)
        copies = []
        for j in range(NUM_CHUNKS):
            sl = pl.ds(j * IDX_CHUNK, IDX_CHUNK)
            copies.append(
                pltpu.async_copy(w_hbm.at[idx_v.at[sl]], rows_v.at[sl], sem)
            )
            copies.append(
                pltpu.async_copy(s_hbm.at[idx_v.at[sl]], scl_v.at[sl], sem)
            )
        for c in copies:
            c.wait()

        def body(r, _):
            row0 = r * LANES
            sv16 = scl_v[pl.ds(row0, LANES)]
            for j in range(LANES):
                sv = lax.broadcast(sv16[j], (LANES,))
                for c in range(MODEL_DIM // LANES):
                    sl = pl.ds(c * LANES, LANES)
                    rows_v[row0 + j, sl] = _quantize(rows_v[row0 + j, sl], sv)
            return 0

        lax.fori_loop(0, B_PER_W // LANES, body, 0)
        pltpu.sync_copy(rows_v, out_hbm.at[pl.ds(base, B_PER_W)])

    return k(x, weights, scales)


def kernel(x, weights, scales):
    return _embed(x.astype(jnp.int32), weights, scales)


# speedup vs baseline: 1.3272x; 1.3272x over previous
import functools

import jax
import jax.numpy as jnp
from jax import lax
from jax.experimental import pallas as pl
from jax.experimental.pallas import tpu as pltpu
from jax.experimental.pallas import tpu_sc as plsc

VOCAB = 100000
MODEL_DIM = 64
BATCH = 16384

NUM_CORES = 2
NUM_SUBCORES = 16
NUM_WORKERS = NUM_CORES * NUM_SUBCORES
B_PER_W = BATCH // NUM_WORKERS
LANES = 16
ROUND_MAGIC = 12582912.0
QMIN = -127.0
QMAX = 127.0


def _quantize(v, sv):
    q = (v + ROUND_MAGIC) - ROUND_MAGIC
    q = jnp.minimum(jnp.maximum(q, QMIN), QMAX)
    return q * sv


def _embed(x, weights, scales):
    mesh = plsc.VectorSubcoreMesh(core_axis_name="c", subcore_axis_name="s")

    @functools.partial(
        pl.kernel,
        mesh=mesh,
        out_type=jax.ShapeDtypeStruct((BATCH, MODEL_DIM), jnp.float32),
        scratch_types=[
            pltpu.VMEM((B_PER_W,), jnp.int32),
            pltpu.VMEM((B_PER_W, MODEL_DIM), jnp.float32),
            pltpu.VMEM((B_PER_W * 8 + 8,), jnp.float32),
            pltpu.SemaphoreType.DMA,
            pltpu.SemaphoreType.DMA,
        ],
        compiler_params=pltpu.CompilerParams(use_tc_tiling_on_sc=True),
    )
    def k(x_hbm, w_hbm, s_hbm, out_hbm, idx_v, rows_v, scl_v, sem, sem2):
        wid = lax.axis_index("s") * NUM_CORES + lax.axis_index("c")
        base = wid * B_PER_W
        pltpu.sync_copy(x_hbm.at[pl.ds(base, B_PER_W)], idx_v)

        def fire(g, _):
            i16 = idx_v[pl.ds(g * LANES, LANES)]
            for j in range(LANES):
                v = i16[j]
                r = g * LANES + j
                pltpu.async_copy(
                    w_hbm.at[pl.ds(v, 1), :], rows_v.at[pl.ds(r, 1), :], sem
                )
                pltpu.async_copy(
                    s_hbm.at[pl.ds((v // 8) * 8, 8)],
                    scl_v.at[pl.ds(r * 8, 8)],
                    sem2,
                )
            return 0

        lax.fori_loop(0, B_PER_W // LANES, fire, 0)
        pltpu.make_async_copy(w_hbm.at[pl.ds(0, B_PER_W), :], rows_v, sem).wait()
        pltpu.make_async_copy(
            s_hbm.at[pl.ds(0, B_PER_W * 8)], scl_v.at[pl.ds(0, B_PER_W * 8)], sem2
        ).wait()

        def body(t, _):
            row0 = t * LANES
            i16 = idx_v[pl.ds(row0, LANES)]
            lanes16 = jnp.bitwise_and(i16, 7)
            for j in range(LANES):
                r = row0 + j
                grp = scl_v[pl.ds(r * 8, LANES)]
                sv = grp.at[jnp.full((LANES,), lanes16[j], jnp.int32)].get(
                    mode="promise_in_bounds"
                )
                for c in range(MODEL_DIM // LANES):
                    sl = pl.ds(c * LANES, LANES)
                    rows_v[r, sl] = _quantize(rows_v[r, sl], sv)
            return 0

        lax.fori_loop(0, B_PER_W // LANES, body, 0)
        pltpu.sync_copy(rows_v, out_hbm.at[pl.ds(base, B_PER_W)])

    return k(x, weights, scales)


def kernel(x, weights, scales):
    return _embed(x.astype(jnp.int32), weights, scales)
